# Initial kernel scaffold; baseline (speedup 1.0000x reference)
import jax
import jax.numpy as jnp
from jax.experimental import pallas as pl


def kernel(boxes, scores):
    raise NotImplementedError("write your pallas kernel here")


# speedup vs baseline: 10.3326x; 10.3326x over previous
import jax
import jax.numpy as jnp
from jax import lax
from jax.experimental import pallas as pl

_NMS_THRESH = 0.5
_SCORE_THRESH = 0.01
_N = 5000
_ROWS = 8
_COLS = 640
_NP = _ROWS * _COLS


def _nms_body(x1_ref, y1_ref, x2_ref, y2_ref, s_ref, keep_ref):
    x1 = x1_ref[...]
    y1 = y1_ref[...]
    x2 = x2_ref[...]
    y2 = y2_ref[...]
    s = s_ref[...]
    idx = (lax.broadcasted_iota(jnp.int32, (_ROWS, _COLS), 0) * _COLS
           + lax.broadcasted_iota(jnp.int32, (_ROWS, _COLS), 1))
    valid = idx < _N
    area = (x2 - x1) * (y2 - y1)

    def cond(carry):
        alive, _ = carry
        return jnp.max(alive) > 0.0

    def body(carry):
        alive, keep = carry
        ms = jnp.where(alive > 0.0, s, -1.0)
        maxv = jnp.max(ms)
        m = jnp.min(jnp.where(ms == maxv, idx, _NP))
        onehot = idx == m
        x1m = jnp.max(jnp.where(onehot, x1, -1.0))
        y1m = jnp.max(jnp.where(onehot, y1, -1.0))
        x2m = jnp.max(jnp.where(onehot, x2, -1.0))
        y2m = jnp.max(jnp.where(onehot, y2, -1.0))
        aream = (x2m - x1m) * (y2m - y1m)
        w = jnp.maximum(0.0, jnp.minimum(x2, x2m) - jnp.maximum(x1, x1m))
        h = jnp.maximum(0.0, jnp.minimum(y2, y2m) - jnp.maximum(y1, y1m))
        inter = w * h
        iou = inter / (area + aream - inter)
        dead = (iou > _NMS_THRESH) | onehot
        keep = jnp.where(onehot, 1.0, keep)
        alive = jnp.where(dead, 0.0, alive)
        return alive, keep

    alive0 = jnp.where(valid, 1.0, 0.0)
    keep0 = jnp.zeros((_ROWS, _COLS), dtype=jnp.float32)
    _, keep = lax.while_loop(cond, body, (alive0, keep0))
    keep_ref[...] = jnp.where(s >= _SCORE_THRESH, keep, 0.0)


def _pad2d(v):
    return jnp.pad(v, (0, _NP - _N)).reshape(_ROWS, _COLS)


@jax.jit
def kernel(boxes, scores):
    cols = [_pad2d(boxes[:, i]) for i in range(4)]
    svec = _pad2d(scores)
    keep2d = pl.pallas_call(
        _nms_body,
        out_shape=jax.ShapeDtypeStruct((_ROWS, _COLS), jnp.float32),
    )(*cols, svec)
    keep = keep2d.reshape(_NP)[:_N]
    kept_boxes = boxes * keep[:, None]
    kept_scores = scores * keep
    return jnp.concatenate([kept_boxes, kept_scores[:, None]], axis=1)


# speedup vs baseline: 18.1052x; 1.7522x over previous
import jax
import jax.numpy as jnp
from jax import lax
from jax.experimental import pallas as pl
from jax.experimental.pallas import tpu as pltpu

_NMS_THRESH = 0.5
_SCORE_THRESH = 0.01
_N = 5000
_ROWS = 8
_COLS = 640
_NP = _ROWS * _COLS


def _nms_body(x1s, y1s, x2s, y2s, x1_ref, y1_ref, x2_ref, y2_ref, s_ref,
              keep_ref):
    x1 = x1_ref[...]
    y1 = y1_ref[...]
    x2 = x2_ref[...]
    y2 = y2_ref[...]
    s = s_ref[...]
    idx = (lax.broadcasted_iota(jnp.int32, (_ROWS, _COLS), 0) * _COLS
           + lax.broadcasted_iota(jnp.int32, (_ROWS, _COLS), 1))
    area = (x2 - x1) * (y2 - y1)

    ms0 = jnp.where(idx < _N, s, -1.0)
    maxv0 = jnp.max(ms0)
    m0 = jnp.min(jnp.where(ms0 == maxv0, idx, _NP))

    def cond(carry):
        _, _, maxv, _ = carry
        return maxv >= _SCORE_THRESH

    def body(carry):
        ms, keep, _, m = carry
        onehot = idx == m
        x1m = x1s[m]
        y1m = y1s[m]
        x2m = x2s[m]
        y2m = y2s[m]
        aream = (x2m - x1m) * (y2m - y1m)
        w = jnp.maximum(0.0, jnp.minimum(x2, x2m) - jnp.maximum(x1, x1m))
        h = jnp.maximum(0.0, jnp.minimum(y2, y2m) - jnp.maximum(y1, y1m))
        inter = w * h
        iou = inter / (area + aream - inter)
        dead = (iou > _NMS_THRESH) | onehot
        keep = jnp.where(onehot, 1.0, keep)
        ms = jnp.where(dead, -1.0, ms)
        maxv = jnp.max(ms)
        m_next = jnp.min(jnp.where(ms == maxv, idx, _NP))
        return ms, keep, maxv, m_next

    keep0 = jnp.zeros((_ROWS, _COLS), dtype=jnp.float32)
    _, keep, _, _ = lax.while_loop(cond, body, (ms0, keep0, maxv0, m0))
    keep_ref[...] = jnp.where(s >= _SCORE_THRESH, keep, 0.0)


def _pad(v):
    return jnp.pad(v, (0, _NP - _N))


@jax.jit
def kernel(boxes, scores):
    flat = [_pad(boxes[:, i]) for i in range(4)]
    cols2d = [v.reshape(_ROWS, _COLS) for v in flat]
    svec = _pad(scores).reshape(_ROWS, _COLS)
    smem_spec = pl.BlockSpec(memory_space=pltpu.SMEM)
    keep2d = pl.pallas_call(
        _nms_body,
        in_specs=[smem_spec] * 4 + [pl.BlockSpec((_ROWS, _COLS),
                                                 lambda: (0, 0))] * 5,
        out_specs=pl.BlockSpec((_ROWS, _COLS), lambda: (0, 0)),
        out_shape=jax.ShapeDtypeStruct((_ROWS, _COLS), jnp.float32),
    )(*flat, *cols2d, svec)
    keep = keep2d.reshape(_NP)[:_N]
    kept_boxes = boxes * keep[:, None]
    kept_scores = scores * keep
    return jnp.concatenate([kept_boxes, kept_scores[:, None]], axis=1)


# speedup vs baseline: 22.3599x; 1.2350x over previous
import jax
import jax.numpy as jnp
from jax import lax
from jax.experimental import pallas as pl
from jax.experimental.pallas import tpu as pltpu

_NMS_THRESH = 0.5
_SCORE_THRESH = 0.01
_N = 5000
_ROWS = 8
_COLS = 640
_NP = _ROWS * _COLS


def _nms_body(x1s, y1s, x2s, y2s, x1_ref, y1_ref, x2_ref, y2_ref, s_ref,
              keep_ref):
    x1 = x1_ref[...]
    y1 = y1_ref[...]
    x2 = x2_ref[...]
    y2 = y2_ref[...]
    s = s_ref[...]
    idx = (lax.broadcasted_iota(jnp.int32, (_ROWS, _COLS), 0) * _COLS
           + lax.broadcasted_iota(jnp.int32, (_ROWS, _COLS), 1))
    idxf = idx.astype(jnp.float32)
    area = (x2 - x1) * (y2 - y1)

    def argmax(ms):
        maxv = jnp.max(ms)
        mf = jnp.min(jnp.where(ms == maxv, idxf, float(_NP)))
        return maxv, mf.astype(jnp.int32)

    def round_(carry):
        ms, keep, maxv, m = carry
        live = maxv >= 0.0
        mc = jnp.minimum(m, _NP - 1)
        onehot = (idx == m) & live
        x1m = x1s[mc]
        y1m = y1s[mc]
        x2m = x2s[mc]
        y2m = y2s[mc]
        aream = (x2m - x1m) * (y2m - y1m)
        w = jnp.maximum(0.0, jnp.minimum(x2, x2m) - jnp.maximum(x1, x1m))
        h = jnp.maximum(0.0, jnp.minimum(y2, y2m) - jnp.maximum(y1, y1m))
        inter = w * h
        iou = inter / (area + aream - inter)
        dead = ((iou > _NMS_THRESH) & live) | onehot
        keep = jnp.where(onehot, 1.0, keep)
        ms = jnp.where(dead, -1.0, ms)
        maxv2, m2 = argmax(ms)
        return ms, keep, maxv2, m2

    def cond(carry):
        return carry[2] >= _SCORE_THRESH

    def body(carry):
        return round_(round_(carry))

    ms0 = jnp.where(idx < _N, s, -1.0)
    keep0 = jnp.zeros((_ROWS, _COLS), dtype=jnp.float32)
    maxv0, m0 = argmax(ms0)
    _, keep, _, _ = lax.while_loop(cond, body, (ms0, keep0, maxv0, m0))
    keep_ref[...] = jnp.where(s >= _SCORE_THRESH, keep, 0.0)


def _pad(v):
    return jnp.pad(v, (0, _NP - _N))


@jax.jit
def kernel(boxes, scores):
    flat = [_pad(boxes[:, i]) for i in range(4)]
    cols2d = [v.reshape(_ROWS, _COLS) for v in flat]
    svec = _pad(scores).reshape(_ROWS, _COLS)
    smem_spec = pl.BlockSpec(memory_space=pltpu.SMEM)
    keep2d = pl.pallas_call(
        _nms_body,
        in_specs=[smem_spec] * 4 + [pl.BlockSpec((_ROWS, _COLS),
                                                 lambda: (0, 0))] * 5,
        out_specs=pl.BlockSpec((_ROWS, _COLS), lambda: (0, 0)),
        out_shape=jax.ShapeDtypeStruct((_ROWS, _COLS), jnp.float32),
    )(*flat, *cols2d, svec)
    keep = keep2d.reshape(_NP)[:_N]
    kept_boxes = boxes * keep[:, None]
    kept_scores = scores * keep
    return jnp.concatenate([kept_boxes, kept_scores[:, None]], axis=1)
